# baseline (speedup 1.0000x reference)
import functools

import jax
import jax.numpy as jnp
from jax import lax
from jax.experimental import pallas as pl
from jax.experimental.pallas import tpu as pltpu, tpu_sc as plsc

_INFO = plsc.get_sparse_core_info()
_NC, _NS = _INFO.num_cores, _INFO.num_subcores
_NW = _NC * _NS
_CH = 8192
_NST = 8
_NBUF = 4
_NG = 4
_NH = 4608


def _gather_body(n_per_w, vocab, ids_hbm, table_hbm, out_hbm,
                 idx_v, rows_v, b0, b1, table_sh, sem, sem_idx,
                 hsems, ssems, gsems):
    bufs = (b0, b1, rows_v.at[pl.ds(0, _CH)], rows_v.at[pl.ds(_CH, _CH)])
    sid = lax.axis_index("s")
    wid = sid * _NC + lax.axis_index("c")
    base = wid * n_per_w
    pltpu.async_copy(ids_hbm.at[pl.ds(base, n_per_w)], idx_v, sem_idx)

    def off(j):
        return jnp.minimum((sid * _NST + j) * _CH, vocab - _CH)

    def hbm_cp(j):
        return pltpu.make_async_copy(
            table_hbm.at[pl.ds(off(j), _CH)], bufs[j % _NBUF],
            hsems.at[j % _NBUF])

    def sp_cp(j):
        return pltpu.make_async_copy(
            bufs[j % _NBUF], table_sh.at[pl.ds(off(j), _CH)],
            ssems.at[j % _NBUF])

    hbm_cp(0).start()
    hbm_cp(1).start()
    n_sp = n_per_w - _NH
    pltpu.make_async_copy(ids_hbm.at[pl.ds(base, n_per_w)], idx_v, sem_idx).wait()
    pre_gather = pltpu.async_copy(
        table_hbm.at[idx_v.at[pl.ds(n_sp, _NH)]], rows_v.at[pl.ds(n_sp, _NH)],
        gsems.at[_NG])
    for j in range(_NST):
        hbm_cp(j).wait()
        sp_cp(j).start()
        if j + 2 < _NST:
            if j >= 2:
                sp_cp(j - 2).wait()
            hbm_cp(j + 2).start()
    for j in range(max(0, _NST - _NBUF), _NST):
        sp_cp(j).wait()
    plsc.subcore_barrier()

    n_sub = n_sp // _NG
    gathers = []
    for g in range(_NG):
        gathers.append(pltpu.async_copy(
            table_sh.at[idx_v.at[pl.ds(g * n_sub, n_sub)]],
            rows_v.at[pl.ds(g * n_sub, n_sub)], gsems.at[g]))
    writes = []
    for g in range(_NG):
        gathers[g].wait()
        writes.append(pltpu.async_copy(
            rows_v.at[pl.ds(g * n_sub, n_sub)],
            out_hbm.at[pl.ds(base + g * n_sub, n_sub)], sem))
    pre_gather.wait()
    writes.append(pltpu.async_copy(
        rows_v.at[pl.ds(n_sp, _NH)], out_hbm.at[pl.ds(base + n_sp, _NH)], sem))
    for w in writes:
        w.wait()


@functools.partial(jax.jit, static_argnames=("n_tokens", "vocab"))
def _gather_sc(token_ids, token_weights, n_tokens, vocab):
    n_per_w = n_tokens // _NW
    assert _NS * _NST * _CH >= vocab
    mesh = plsc.VectorSubcoreMesh(core_axis_name="c", subcore_axis_name="s")
    k = pl.kernel(
        _gather_body_bound(n_per_w, vocab),
        mesh=mesh,
        out_type=jax.ShapeDtypeStruct((n_tokens,), jnp.float32),
        scratch_types=[
            pltpu.VMEM((n_per_w,), jnp.int32),
            pltpu.VMEM((n_per_w,), jnp.float32),
            pltpu.VMEM((_CH,), jnp.float32),
            pltpu.VMEM((_CH,), jnp.float32),
            pltpu.VMEM_SHARED((vocab,), jnp.float32),
            pltpu.SemaphoreType.DMA,
            pltpu.SemaphoreType.DMA,
            pltpu.SemaphoreType.DMA((_NBUF,)),
            pltpu.SemaphoreType.DMA((_NBUF,)),
            pltpu.SemaphoreType.DMA((_NG + 1,)),
        ],
    )
    return k(token_ids, token_weights)


def _gather_body_bound(n_per_w, vocab):
    return functools.partial(_gather_body, n_per_w, vocab)


def kernel(token_ids, token_weights):
    n_tokens = token_ids.shape[0]
    vocab = token_weights.shape[0]
    return _gather_sc(token_ids.astype(jnp.int32), token_weights, n_tokens, vocab)


# speedup vs baseline: 1.0542x; 1.0542x over previous
import functools

import jax
import jax.numpy as jnp
from jax import lax
from jax.experimental import pallas as pl
from jax.experimental.pallas import tpu as pltpu, tpu_sc as plsc

_INFO = plsc.get_sparse_core_info()
_NC, _NS = _INFO.num_cores, _INFO.num_subcores
_NW = _NC * _NS
_CH = 8192
_NST = 8
_NBUF = 4
_NG = 4


def _gather_body(n_per_w, vocab, ids_hbm, table_hbm, out_hbm,
                 idx_v, rows_v, b0, b1, table_sh, sem, sem_idx,
                 hsems, ssems, gsems):
    bufs = (rows_v.at[pl.ds(0, _CH)], rows_v.at[pl.ds(_CH, _CH)], b0, b1)
    sid = lax.axis_index("s")
    wid = sid * _NC + lax.axis_index("c")
    base = wid * n_per_w
    pltpu.async_copy(ids_hbm.at[pl.ds(base, n_per_w)], idx_v, sem_idx)

    t_span = -(-vocab // (8 * _NS)) * 8
    last_sz = ((t_span - (_NST - 1) * _CH) + 7) & ~7

    def sz(j):
        return _CH if j < _NST - 1 else last_sz

    def off(j):
        return jnp.minimum(sid * t_span + j * _CH, vocab - sz(j))

    def buf(j):
        b = bufs[j % _NBUF]
        return b if sz(j) == _CH else b.at[pl.ds(0, sz(j))]

    def hbm_cp(j):
        return pltpu.make_async_copy(
            table_hbm.at[pl.ds(off(j), sz(j))], buf(j), hsems.at[j % _NBUF])

    def sp_cp(j):
        return pltpu.make_async_copy(
            buf(j), table_sh.at[pl.ds(off(j), sz(j))], ssems.at[j % _NBUF])

    hbm_cp(0).start()
    hbm_cp(1).start()
    for j in range(_NST):
        hbm_cp(j).wait()
        sp_cp(j).start()
        if j + 2 < _NST:
            if j >= 2:
                sp_cp(j - 2).wait()
            hbm_cp(j + 2).start()
    for j in range(max(0, _NST - _NBUF), _NST):
        sp_cp(j).wait()
    plsc.subcore_barrier()

    pltpu.make_async_copy(ids_hbm.at[pl.ds(base, n_per_w)], idx_v, sem_idx).wait()
    n_sub = n_per_w // _NG
    gathers = []
    for g in range(_NG):
        gathers.append(pltpu.async_copy(
            table_sh.at[idx_v.at[pl.ds(g * n_sub, n_sub)]],
            rows_v.at[pl.ds(g * n_sub, n_sub)], gsems.at[g]))
    writes = []
    for g in range(_NG):
        gathers[g].wait()
        writes.append(pltpu.async_copy(
            rows_v.at[pl.ds(g * n_sub, n_sub)],
            out_hbm.at[pl.ds(base + g * n_sub, n_sub)], sem))
    for w in writes:
        w.wait()


@functools.partial(jax.jit, static_argnames=("n_tokens", "vocab"))
def _gather_sc(token_ids, token_weights, n_tokens, vocab):
    n_per_w = n_tokens // _NW
    assert _NS * _NST * _CH >= vocab
    mesh = plsc.VectorSubcoreMesh(core_axis_name="c", subcore_axis_name="s")
    k = pl.kernel(
        _gather_body_bound(n_per_w, vocab),
        mesh=mesh,
        out_type=jax.ShapeDtypeStruct((n_tokens,), jnp.float32),
        scratch_types=[
            pltpu.VMEM((n_per_w,), jnp.int32),
            pltpu.VMEM((n_per_w,), jnp.float32),
            pltpu.VMEM((_CH,), jnp.float32),
            pltpu.VMEM((_CH,), jnp.float32),
            pltpu.VMEM_SHARED((vocab,), jnp.float32),
            pltpu.SemaphoreType.DMA,
            pltpu.SemaphoreType.DMA,
            pltpu.SemaphoreType.DMA((_NBUF,)),
            pltpu.SemaphoreType.DMA((_NBUF,)),
            pltpu.SemaphoreType.DMA((_NG,)),
        ],
    )
    return k(token_ids, token_weights)


def _gather_body_bound(n_per_w, vocab):
    return functools.partial(_gather_body, n_per_w, vocab)


def kernel(token_ids, token_weights):
    n_tokens = token_ids.shape[0]
    vocab = token_weights.shape[0]
    return _gather_sc(token_ids.astype(jnp.int32), token_weights, n_tokens, vocab)
